# baseline (speedup 1.0000x reference)
import functools

import jax
import jax.numpy as jnp
from jax import lax
from jax.experimental import pallas as pl
from jax.experimental.pallas import tpu as pltpu
from jax.experimental.pallas import tpu_sc as plsc

B = 16384
D = 64
NNEG = 20
IPAD = 32
CB = 32
KV = D // 16
NSLAB = CB * NNEG // 128

_GDN = lax.GatherDimensionNumbers(
    offset_dims=(), collapsed_slice_dims=(0,), start_index_map=(0,))


def _lane_perm(x, idx):
    return lax.gather(x, idx[:, None], _GDN, slice_sizes=(1,),
                      mode=lax.GatherScatterMode.PROMISE_IN_BOUNDS)


@functools.cache
def _build_sc_scores(nc: int, ns: int):
    nw = nc * ns
    bpw = B // nw
    nchunk = bpw // CB
    mesh = plsc.VectorSubcoreMesh(core_axis_name="c", subcore_axis_name="s")

    def body(uid, pid, nid, utab, itab, out,
             idx_u, idx_p, idx_n, u_rows, p_rows, n_rows, obuf, sem):
        wid = lax.axis_index("s") * nc + lax.axis_index("c")
        lane = lax.iota(jnp.int32, 16)

        @pl.loop(0, nchunk)
        def _chunk(ci):
            g = wid * nchunk + ci
            base = g * CB
            pltpu.sync_copy(uid.at[pl.ds(base, CB)], idx_u)
            pltpu.sync_copy(pid.at[pl.ds(base, CB)], idx_p)
            for j in range(NSLAB):
                pltpu.sync_copy(nid.at[g * NSLAB + j], idx_n.at[j])
            cps = [pltpu.async_copy(utab.at[idx_u], u_rows, sem),
                   pltpu.async_copy(itab.at[idx_p], p_rows, sem)]
            for j in range(NSLAB):
                cps.append(pltpu.async_copy(
                    itab.at[idx_n.at[j]], n_rows.at[pl.ds(j * 128, 128)], sem))
            for cp in cps:
                cp.wait()

            @pl.loop(0, CB)
            def _row(b):
                u = [u_rows[b, pl.ds(k * 16, 16)] for k in range(KV)]

                def dot(ref, r):
                    acc = u[0] * ref[r, pl.ds(0, 16)]
                    for k in range(1, KV):
                        acc = acc + u[k] * ref[r, pl.ds(k * 16, 16)]
                    for s in (8, 4, 2, 1):
                        acc = acc + _lane_perm(acc, lane ^ s)
                    return acc

                zero = jnp.zeros((16,), jnp.float32)
                r0 = jnp.where(lane == 0, dot(p_rows, b), zero)
                r1 = zero
                for n in range(NNEG):
                    col = n + 1
                    total = dot(n_rows, b * NNEG + n)
                    if col < 16:
                        r0 = jnp.where(lane == col, total, r0)
                    else:
                        r1 = jnp.where(lane == col - 16, total, r1)
                obuf[pl.ds(b * IPAD, 16)] = r0
                obuf[pl.ds(b * IPAD + 16, 16)] = r1

            pltpu.sync_copy(obuf, out.at[pl.ds(base * IPAD, CB * IPAD)])

    return pl.kernel(
        body,
        out_type=jax.ShapeDtypeStruct((B * IPAD,), jnp.float32),
        mesh=mesh,
        compiler_params=pltpu.CompilerParams(use_tc_tiling_on_sc=False),
        scratch_types=[
            pltpu.VMEM((CB,), jnp.int32),
            pltpu.VMEM((CB,), jnp.int32),
            pltpu.VMEM((NSLAB, 128), jnp.int32),
            pltpu.VMEM((CB, D), jnp.float32),
            pltpu.VMEM((CB, D), jnp.float32),
            pltpu.VMEM((CB * NNEG, D), jnp.float32),
            pltpu.VMEM((CB * IPAD,), jnp.float32),
            pltpu.SemaphoreType.DMA,
        ],
    )


def _loss_body(s_ref, o_ref):
    x = s_ref[...]
    col = lax.broadcasted_iota(jnp.int32, (B, IPAD), 1)
    pos = jnp.sum(jnp.where(col == 0, x, 0.0), axis=1, keepdims=True)
    lval = jnp.log(jax.nn.sigmoid(pos - x) + 1e-10)
    valid = (col >= 1) & (col <= NNEG)
    o_ref[0, 0] = -jnp.sum(jnp.where(valid, lval, 0.0)) * (1.0 / (B * NNEG))


_loss = pl.pallas_call(
    _loss_body,
    out_shape=jax.ShapeDtypeStruct((1, 1), jnp.float32),
    out_specs=pl.BlockSpec(memory_space=pltpu.SMEM))


def kernel(user_ids, pos_item_ids, neg_item_ids, user_table, item_table):
    info = plsc.get_sparse_core_info()
    sc_scores = _build_sc_scores(info.num_cores, info.num_subcores)
    nid = neg_item_ids.reshape(B * NNEG // 128, 128)
    flat = sc_scores(user_ids, pos_item_ids, nid, user_table, item_table)
    return _loss(flat.reshape(B, IPAD))[0, 0]
